# baseline (speedup 1.0000x reference)
import jax
import jax.numpy as jnp
from jax.experimental import pallas as pl
from jax.experimental.pallas import tpu as pltpu

B, N, M = 16, 1024, 256
C1, C2 = 256, 256
OUT0, OUT1 = 256, 256
NROWS = B * N
ROWBLK = 8192
BPB = 4
EPS_BN = 1e-5
EPS_D = 1e-8


def _k1_one(x1t, x2p, p1, p2, w0at_ref, w0bt_ref, b0_ref):
    x1sq = jnp.sum(x1t * x1t, axis=0, keepdims=True)
    x2sq = jnp.sum(x2p * x2p, axis=1, keepdims=True)
    cross = jax.lax.dot_general(
        x2p, x1t, (((1,), (0,)), ((), ())),
        preferred_element_type=jnp.float32,
        precision=jax.lax.Precision.DEFAULT)
    d2 = jnp.maximum(x2sq + x1sq - 2.0 * cross, 0.0)

    big = jnp.float32(3.4e38)
    iota_f = jax.lax.broadcasted_iota(jnp.int32, (M, N), 0).astype(jnp.float32)
    s_t = jnp.zeros((M, N), jnp.float32)
    recip_sum = jnp.zeros((1, N), jnp.float32)
    for _ in range(3):
        mval = jnp.min(d2, axis=0, keepdims=True)
        sel = jnp.min(jnp.where(d2 == mval, iota_f, big),
                      axis=0, keepdims=True)
        hit = iota_f == sel
        r = 1.0 / (mval + EPS_D)
        recip_sum = recip_sum + r
        s_t = jnp.where(hit, r, s_t)
        d2 = jnp.where(hit, big, d2)
    s_t = s_t * (1.0 / recip_sum)

    z = jax.lax.dot_general(
        p2, w0bt_ref[...], (((1,), (0,)), ((), ())),
        preferred_element_type=jnp.float32)
    y0 = (jax.lax.dot_general(p1, w0at_ref[...],
                              (((1,), (0,)), ((), ())),
                              preferred_element_type=jnp.float32)
          + jax.lax.dot_general(s_t, z, (((0,), (0,)), ((), ())),
                                preferred_element_type=jnp.float32)
          + b0_ref[...])
    return y0


def _k1_body(x1t_ref, x2p_ref, p1_ref, p2_ref, w0at_ref, w0bt_ref, b0_ref,
             y0_ref, stats_ref):
    b = pl.program_id(0)

    @pl.when(b == 0)
    def _init():
        stats_ref[...] = jnp.zeros_like(stats_ref)

    part = None
    for i in range(BPB):
        y0 = _k1_one(x1t_ref[i], x2p_ref[i], p1_ref[i], p2_ref[i],
                     w0at_ref, w0bt_ref, b0_ref)
        y0_ref[i] = y0.astype(jnp.bfloat16)
        p = jnp.concatenate(
            [jnp.sum(y0, axis=0, keepdims=True),
             jnp.sum(y0 * y0, axis=0, keepdims=True)], axis=0)
        part = p if part is None else part + p
    stats_ref[...] += part


def _k2_body(y0_ref, stats0_ref, w1t_ref, b1_ref, g0_ref, beta0_ref,
             y1_ref, stats1_ref):
    i = pl.program_id(0)
    inv_n = jnp.float32(1.0 / NROWS)
    mean = stats0_ref[0:1, :] * inv_n
    var = stats0_ref[1:2, :] * inv_n - mean * mean
    scale = g0_ref[...] * jax.lax.rsqrt(var + EPS_BN)
    shift = beta0_ref[...] - mean * scale
    h = jnp.maximum(y0_ref[...].astype(jnp.float32) * scale + shift, 0.0)
    y1 = jax.lax.dot_general(h, w1t_ref[...], (((1,), (0,)), ((), ())),
                             preferred_element_type=jnp.float32) + b1_ref[...]
    y1_ref[...] = y1.astype(jnp.bfloat16)

    @pl.when(i == 0)
    def _init():
        stats1_ref[...] = jnp.zeros_like(stats1_ref)

    stats1_ref[...] += jnp.concatenate(
        [jnp.sum(y1, axis=0, keepdims=True),
         jnp.sum(y1 * y1, axis=0, keepdims=True)], axis=0)


def _k3_body(y1_ref, stats1_ref, g1_ref, beta1_ref, out_ref):
    inv_n = jnp.float32(1.0 / NROWS)
    mean = stats1_ref[0:1, :] * inv_n
    var = stats1_ref[1:2, :] * inv_n - mean * mean
    scale = g1_ref[...] * jax.lax.rsqrt(var + EPS_BN)
    shift = beta1_ref[...] - mean * scale
    out_ref[...] = jnp.maximum(
        y1_ref[...].astype(jnp.float32) * scale + shift, 0.0)


@jax.jit
def kernel(xyz1, xyz2, points1, points2, W0, b0, g0, beta0, W1, b1, g1, beta1):
    f32 = jnp.float32
    x1t = jnp.pad(xyz1, ((0, 0), (0, 0), (0, 5))).transpose(0, 2, 1)
    x2p = jnp.pad(xyz2, ((0, 0), (0, 0), (0, 5)))
    w0t = W0.T
    w0at, w0bt = w0t[:C1], w0t[C1:]
    w1t = W1.T
    row = lambda v: v.reshape(1, -1)

    y0, stats0 = pl.pallas_call(
        _k1_body,
        grid=(B // BPB,),
        in_specs=[
            pl.BlockSpec((BPB, 8, N), lambda b: (b, 0, 0)),
            pl.BlockSpec((BPB, M, 8), lambda b: (b, 0, 0)),
            pl.BlockSpec((BPB, N, C1), lambda b: (b, 0, 0)),
            pl.BlockSpec((BPB, M, C2), lambda b: (b, 0, 0)),
            pl.BlockSpec((C1, OUT0), lambda b: (0, 0)),
            pl.BlockSpec((C2, OUT0), lambda b: (0, 0)),
            pl.BlockSpec((1, OUT0), lambda b: (0, 0)),
        ],
        out_specs=[
            pl.BlockSpec((BPB, N, OUT0), lambda b: (b, 0, 0)),
            pl.BlockSpec((2, OUT0), lambda b: (0, 0)),
        ],
        out_shape=[
            jax.ShapeDtypeStruct((B, N, OUT0), jnp.bfloat16),
            jax.ShapeDtypeStruct((2, OUT0), f32),
        ],
    )(x1t, x2p, points1, points2, w0at, w0bt, row(b0))

    y0f = y0.reshape(NROWS, OUT0)
    nblk = NROWS // ROWBLK
    y1, stats1 = pl.pallas_call(
        _k2_body,
        grid=(nblk,),
        in_specs=[
            pl.BlockSpec((ROWBLK, OUT0), lambda i: (i, 0)),
            pl.BlockSpec((2, OUT0), lambda i: (0, 0)),
            pl.BlockSpec((OUT0, OUT1), lambda i: (0, 0)),
            pl.BlockSpec((1, OUT1), lambda i: (0, 0)),
            pl.BlockSpec((1, OUT0), lambda i: (0, 0)),
            pl.BlockSpec((1, OUT0), lambda i: (0, 0)),
        ],
        out_specs=[
            pl.BlockSpec((ROWBLK, OUT1), lambda i: (i, 0)),
            pl.BlockSpec((2, OUT1), lambda i: (0, 0)),
        ],
        out_shape=[
            jax.ShapeDtypeStruct((NROWS, OUT1), jnp.bfloat16),
            jax.ShapeDtypeStruct((2, OUT1), f32),
        ],
    )(y0f, stats0, w1t, row(b1), row(g0), row(beta0))

    out = pl.pallas_call(
        _k3_body,
        grid=(nblk,),
        in_specs=[
            pl.BlockSpec((ROWBLK, OUT1), lambda i: (i, 0)),
            pl.BlockSpec((2, OUT1), lambda i: (0, 0)),
            pl.BlockSpec((1, OUT1), lambda i: (0, 0)),
            pl.BlockSpec((1, OUT1), lambda i: (0, 0)),
        ],
        out_specs=pl.BlockSpec((ROWBLK, OUT1), lambda i: (i, 0)),
        out_shape=jax.ShapeDtypeStruct((NROWS, OUT1), f32),
    )(y1, stats1, row(g1), row(beta1))

    return out.reshape(B, N, OUT1)


# speedup vs baseline: 1.0190x; 1.0190x over previous
import jax
import jax.numpy as jnp
from jax.experimental import pallas as pl
from jax.experimental.pallas import tpu as pltpu

B, N, M = 16, 1024, 256
C1, C2 = 256, 256
OUT0, OUT1 = 256, 256
NROWS = B * N
ROWBLK = 8192
BPB = 2
EPS_BN = 1e-5
EPS_D = 1e-8


def _k1_one(x1t, x2p, p1, p2, w0at_ref, w0bt_ref, b0_ref):
    x1sq = jnp.sum(x1t * x1t, axis=0, keepdims=True)
    x2sq = jnp.sum(x2p * x2p, axis=1, keepdims=True)
    cross = jax.lax.dot_general(
        x2p, x1t, (((1,), (0,)), ((), ())),
        preferred_element_type=jnp.float32,
        precision=jax.lax.Precision.DEFAULT)
    d2 = jnp.maximum(x2sq + x1sq - 2.0 * cross, 0.0)

    big = jnp.float32(3.4e38)
    iota_f = jax.lax.broadcasted_iota(jnp.int32, (M, N), 0).astype(jnp.float32)
    s_t = jnp.zeros((M, N), jnp.float32)
    recip_sum = jnp.zeros((1, N), jnp.float32)
    for _ in range(3):
        mval = jnp.min(d2, axis=0, keepdims=True)
        sel = jnp.min(jnp.where(d2 == mval, iota_f, big),
                      axis=0, keepdims=True)
        hit = iota_f == sel
        r = 1.0 / (mval + EPS_D)
        recip_sum = recip_sum + r
        s_t = jnp.where(hit, r, s_t)
        d2 = jnp.where(hit, big, d2)
    s_t = s_t * (1.0 / recip_sum)

    z = jax.lax.dot_general(
        p2, w0bt_ref[...], (((1,), (0,)), ((), ())),
        preferred_element_type=jnp.float32)
    y0 = (jax.lax.dot_general(p1, w0at_ref[...],
                              (((1,), (0,)), ((), ())),
                              preferred_element_type=jnp.float32)
          + jax.lax.dot_general(s_t, z, (((0,), (0,)), ((), ())),
                                preferred_element_type=jnp.float32)
          + b0_ref[...])
    return y0


def _k1_body(x1t_ref, x2p_ref, p1_ref, p2_ref, w0at_ref, w0bt_ref, b0_ref,
             y0_ref, stats_ref):
    b = pl.program_id(0)

    @pl.when(b == 0)
    def _init():
        stats_ref[...] = jnp.zeros_like(stats_ref)

    part = None
    for i in range(BPB):
        y0 = _k1_one(x1t_ref[i], x2p_ref[i], p1_ref[i], p2_ref[i],
                     w0at_ref, w0bt_ref, b0_ref)
        y0_ref[i] = y0.astype(jnp.bfloat16)
        p = jnp.concatenate(
            [jnp.sum(y0, axis=0, keepdims=True),
             jnp.sum(y0 * y0, axis=0, keepdims=True)], axis=0)
        part = p if part is None else part + p
    stats_ref[...] += part


def _k2_body(y0_ref, stats0_ref, w1t_ref, b1_ref, g0_ref, beta0_ref,
             y1_ref, stats1_ref):
    i = pl.program_id(0)
    inv_n = jnp.float32(1.0 / NROWS)
    mean = stats0_ref[0:1, :] * inv_n
    var = stats0_ref[1:2, :] * inv_n - mean * mean
    scale = g0_ref[...] * jax.lax.rsqrt(var + EPS_BN)
    shift = beta0_ref[...] - mean * scale
    h = jnp.maximum(y0_ref[...].astype(jnp.float32) * scale + shift, 0.0)
    y1 = jax.lax.dot_general(h, w1t_ref[...], (((1,), (0,)), ((), ())),
                             preferred_element_type=jnp.float32) + b1_ref[...]
    y1_ref[...] = y1.astype(jnp.bfloat16)

    @pl.when(i == 0)
    def _init():
        stats1_ref[...] = jnp.zeros_like(stats1_ref)

    stats1_ref[...] += jnp.concatenate(
        [jnp.sum(y1, axis=0, keepdims=True),
         jnp.sum(y1 * y1, axis=0, keepdims=True)], axis=0)


def _k3_body(y1_ref, stats1_ref, g1_ref, beta1_ref, out_ref):
    inv_n = jnp.float32(1.0 / NROWS)
    mean = stats1_ref[0:1, :] * inv_n
    var = stats1_ref[1:2, :] * inv_n - mean * mean
    scale = g1_ref[...] * jax.lax.rsqrt(var + EPS_BN)
    shift = beta1_ref[...] - mean * scale
    out_ref[...] = jnp.maximum(
        y1_ref[...].astype(jnp.float32) * scale + shift, 0.0)


@jax.jit
def kernel(xyz1, xyz2, points1, points2, W0, b0, g0, beta0, W1, b1, g1, beta1):
    f32 = jnp.float32
    x1t = jnp.pad(xyz1, ((0, 0), (0, 0), (0, 5))).transpose(0, 2, 1)
    x2p = jnp.pad(xyz2, ((0, 0), (0, 0), (0, 5)))
    w0t = W0.T
    w0at, w0bt = w0t[:C1], w0t[C1:]
    w1t = W1.T
    row = lambda v: v.reshape(1, -1)

    y0, stats0 = pl.pallas_call(
        _k1_body,
        grid=(B // BPB,),
        in_specs=[
            pl.BlockSpec((BPB, 8, N), lambda b: (b, 0, 0)),
            pl.BlockSpec((BPB, M, 8), lambda b: (b, 0, 0)),
            pl.BlockSpec((BPB, N, C1), lambda b: (b, 0, 0)),
            pl.BlockSpec((BPB, M, C2), lambda b: (b, 0, 0)),
            pl.BlockSpec((C1, OUT0), lambda b: (0, 0)),
            pl.BlockSpec((C2, OUT0), lambda b: (0, 0)),
            pl.BlockSpec((1, OUT0), lambda b: (0, 0)),
        ],
        out_specs=[
            pl.BlockSpec((BPB, N, OUT0), lambda b: (b, 0, 0)),
            pl.BlockSpec((2, OUT0), lambda b: (0, 0)),
        ],
        out_shape=[
            jax.ShapeDtypeStruct((B, N, OUT0), jnp.bfloat16),
            jax.ShapeDtypeStruct((2, OUT0), f32),
        ],
    )(x1t, x2p, points1, points2, w0at, w0bt, row(b0))

    y0f = y0.reshape(NROWS, OUT0)
    nblk = NROWS // ROWBLK
    y1, stats1 = pl.pallas_call(
        _k2_body,
        grid=(nblk,),
        in_specs=[
            pl.BlockSpec((ROWBLK, OUT0), lambda i: (i, 0)),
            pl.BlockSpec((2, OUT0), lambda i: (0, 0)),
            pl.BlockSpec((OUT0, OUT1), lambda i: (0, 0)),
            pl.BlockSpec((1, OUT1), lambda i: (0, 0)),
            pl.BlockSpec((1, OUT0), lambda i: (0, 0)),
            pl.BlockSpec((1, OUT0), lambda i: (0, 0)),
        ],
        out_specs=[
            pl.BlockSpec((ROWBLK, OUT1), lambda i: (i, 0)),
            pl.BlockSpec((2, OUT1), lambda i: (0, 0)),
        ],
        out_shape=[
            jax.ShapeDtypeStruct((NROWS, OUT1), jnp.bfloat16),
            jax.ShapeDtypeStruct((2, OUT1), f32),
        ],
    )(y0f, stats0, w1t, row(b1), row(g0), row(beta0))

    out = pl.pallas_call(
        _k3_body,
        grid=(nblk,),
        in_specs=[
            pl.BlockSpec((ROWBLK, OUT1), lambda i: (i, 0)),
            pl.BlockSpec((2, OUT1), lambda i: (0, 0)),
            pl.BlockSpec((1, OUT1), lambda i: (0, 0)),
            pl.BlockSpec((1, OUT1), lambda i: (0, 0)),
        ],
        out_specs=pl.BlockSpec((ROWBLK, OUT1), lambda i: (i, 0)),
        out_shape=jax.ShapeDtypeStruct((NROWS, OUT1), f32),
    )(y1, stats1, row(g1), row(beta1))

    return out.reshape(B, N, OUT1)
